# Initial kernel scaffold; baseline (speedup 1.0000x reference)
import functools

import jax
import jax.numpy as jnp
from jax import lax
from jax.experimental import pallas as pl
from jax.experimental.pallas import tpu as pltpu
from jax.experimental.pallas import tpu_sc as plsc

LANES = 16
NC = 2
NS = 16
NW = NC * NS


@functools.lru_cache(maxsize=None)
def _build(n_tok, vocab, emb, n_pos, pos_w, n_bio, bio_w):
    out_w = emb + pos_w + bio_w
    chunk = 512
    assert n_tok % (NW * chunk) == 0
    tok_per_w = n_tok // NW
    n_chunks = tok_per_w // chunk

    mesh = plsc.VectorSubcoreMesh(core_axis_name="c", subcore_axis_name="s")

    @functools.partial(
        pl.kernel,
        mesh=mesh,
        out_type=jax.ShapeDtypeStruct((n_tok, out_w), jnp.float32),
        scratch_types=[
            pltpu.VMEM((chunk,), jnp.int32),
            pltpu.VMEM((chunk,), jnp.int32),
            pltpu.VMEM((chunk,), jnp.int32),
            pltpu.VMEM((n_pos, pos_w), jnp.float32),
            pltpu.VMEM((n_bio, bio_w), jnp.float32),
            pltpu.VMEM((chunk, emb), jnp.float32),
            pltpu.VMEM((chunk, out_w), jnp.float32),
            pltpu.SemaphoreType.DMA,
        ],
    )
    def sc_kernel(sent_hbm, f0_hbm, f1_hbm, pos_hbm, bio_hbm, table_hbm,
                  out_hbm, idx_v, f0_v, f1_v, pos_v, bio_v, word_v, out_v,
                  gsem):
        wid = lax.axis_index("s") * NC + lax.axis_index("c")
        base = wid * tok_per_w
        pltpu.sync_copy(pos_hbm, pos_v)
        pltpu.sync_copy(bio_hbm, bio_v)
        iota = lax.iota(jnp.int32, LANES)

        def chunk_body(j, carry):
            cb = base + j * chunk
            pltpu.sync_copy(sent_hbm.at[pl.ds(cb, chunk)], idx_v)
            gather = pltpu.async_copy(table_hbm.at[idx_v], word_v, gsem)
            pltpu.sync_copy(f0_hbm.at[pl.ds(cb, chunk)], f0_v)
            pltpu.sync_copy(f1_hbm.at[pl.ds(cb, chunk)], f1_v)

            def fill_body(t, c):
                rows = t * LANES + iota
                f0 = f0_v[pl.ds(t * LANES, LANES)]
                f1 = f1_v[pl.ds(t * LANES, LANES)]
                for col in range(pos_w):
                    vals = plsc.load_gather(
                        pos_v, [f0, jnp.full((LANES,), col, jnp.int32)])
                    plsc.store_scatter(
                        out_v,
                        [rows, jnp.full((LANES,), emb + col, jnp.int32)],
                        vals)
                for col in range(bio_w):
                    vals = plsc.load_gather(
                        bio_v, [f1, jnp.full((LANES,), col, jnp.int32)])
                    plsc.store_scatter(
                        out_v,
                        [rows, jnp.full((LANES,), emb + pos_w + col,
                                        jnp.int32)],
                        vals)
                return c

            lax.fori_loop(0, chunk // LANES, fill_body, 0)
            gather.wait()

            def copy_body(t, c):
                for k in range(emb // LANES):
                    out_v[t, pl.ds(k * LANES, LANES)] = (
                        word_v[t, pl.ds(k * LANES, LANES)])
                return c

            lax.fori_loop(0, chunk, copy_body, 0)
            pltpu.sync_copy(out_v, out_hbm.at[pl.ds(cb, chunk)])
            return carry

        lax.fori_loop(0, n_chunks, chunk_body, 0)

    return sc_kernel


def kernel(sentence, features, embedding_matrix, pos_table, bio_table):
    b, l = sentence.shape
    vocab, emb = embedding_matrix.shape
    n_pos, pos_w = pos_table.shape
    n_bio, bio_w = bio_table.shape
    n_tok = b * l
    sent = sentence.reshape(n_tok)
    f0 = features[..., 0].reshape(n_tok)
    f1 = features[..., 1].reshape(n_tok)
    fn = _build(n_tok, vocab, emb, n_pos, pos_w, n_bio, bio_w)
    out = fn(sent, f0, f1, pos_table, bio_table, embedding_matrix)
    return out.reshape(b, l, emb + pos_w + bio_w)


# speedup vs baseline: 2.7595x; 2.7595x over previous
import functools

import jax
import jax.numpy as jnp
from jax import lax
from jax.experimental import pallas as pl
from jax.experimental.pallas import tpu as pltpu
from jax.experimental.pallas import tpu_sc as plsc

LANES = 16
NC = 2
NS = 16
NW = NC * NS


@functools.lru_cache(maxsize=None)
def _build(n_tok, vocab, emb, n_pos, pos_w, n_bio, bio_w):
    out_w = emb + pos_w + bio_w
    chunk = 512
    assert n_tok % (NW * chunk) == 0
    tok_per_w = n_tok // NW
    n_chunks = tok_per_w // chunk

    mesh = plsc.VectorSubcoreMesh(core_axis_name="c", subcore_axis_name="s")

    @functools.partial(
        pl.kernel,
        mesh=mesh,
        out_type=jax.ShapeDtypeStruct((n_tok * out_w,), jnp.float32),
        compiler_params=pltpu.CompilerParams(
            needs_layout_passes=False, use_tc_tiling_on_sc=False),
        scratch_types=[
            pltpu.VMEM((chunk,), jnp.int32),
            pltpu.VMEM((chunk,), jnp.int32),
            pltpu.VMEM((chunk,), jnp.int32),
            pltpu.VMEM((n_pos * pos_w,), jnp.float32),
            pltpu.VMEM((n_bio * bio_w,), jnp.float32),
            pltpu.VMEM((chunk, emb), jnp.float32),
            pltpu.VMEM((chunk * out_w,), jnp.float32),
            pltpu.SemaphoreType.DMA,
        ],
    )
    def sc_kernel(sent_hbm, f0_hbm, f1_hbm, pos_hbm, bio_hbm, table_hbm,
                  out_hbm, idx_v, f0_v, f1_v, pos_v, bio_v, word_v, out_v,
                  gsem):
        wid = lax.axis_index("s") * NC + lax.axis_index("c")
        base = wid * tok_per_w
        pltpu.sync_copy(pos_hbm, pos_v)
        pltpu.sync_copy(bio_hbm, bio_v)
        iota = lax.iota(jnp.int32, LANES)

        def chunk_body(j, carry):
            cb = base + j * chunk
            pltpu.sync_copy(sent_hbm.at[pl.ds(cb, chunk)], idx_v)
            gather = pltpu.async_copy(table_hbm.at[idx_v], word_v, gsem)
            pltpu.sync_copy(f0_hbm.at[pl.ds(cb, chunk)], f0_v)
            pltpu.sync_copy(f1_hbm.at[pl.ds(cb, chunk)], f1_v)

            def fill_body(t, c):
                rowbase = (t * LANES + iota) * out_w + emb
                f0s = f0_v[pl.ds(t * LANES, LANES)] * pos_w
                f1s = f1_v[pl.ds(t * LANES, LANES)] * bio_w
                for col in range(pos_w):
                    vals = plsc.load_gather(pos_v, [f0s + col])
                    plsc.store_scatter(out_v, [rowbase + col], vals)
                for col in range(bio_w):
                    vals = plsc.load_gather(bio_v, [f1s + col])
                    plsc.store_scatter(out_v, [rowbase + (pos_w + col)],
                                       vals)
                return c

            lax.fori_loop(0, chunk // LANES, fill_body, 0)
            gather.wait()

            def copy_body(t, c):
                for k in range(emb // LANES):
                    out_v[pl.ds(t * out_w + k * LANES, LANES)] = (
                        word_v[t, pl.ds(k * LANES, LANES)])
                return c

            lax.fori_loop(0, chunk, copy_body, 0)
            pltpu.sync_copy(out_v,
                            out_hbm.at[pl.ds(cb * out_w, chunk * out_w)])
            return carry

        lax.fori_loop(0, n_chunks, chunk_body, 0)

    return sc_kernel


def kernel(sentence, features, embedding_matrix, pos_table, bio_table):
    b, l = sentence.shape
    vocab, emb = embedding_matrix.shape
    n_pos, pos_w = pos_table.shape
    n_bio, bio_w = bio_table.shape
    n_tok = b * l
    sent = sentence.reshape(n_tok)
    f0 = features[..., 0].reshape(n_tok)
    f1 = features[..., 1].reshape(n_tok)
    fn = _build(n_tok, vocab, emb, n_pos, pos_w, n_bio, bio_w)
    out = fn(sent, f0, f1, pos_table.reshape(n_pos * pos_w),
             bio_table.reshape(n_bio * bio_w), embedding_matrix)
    return out.reshape(b, l, emb + pos_w + bio_w)


# speedup vs baseline: 3.7329x; 1.3527x over previous
import functools

import jax
import jax.numpy as jnp
from jax import lax
from jax.experimental import pallas as pl
from jax.experimental.pallas import tpu as pltpu
from jax.experimental.pallas import tpu_sc as plsc

LANES = 16
NC = 2
NS = 16
NW = NC * NS
NBUF = 2


@functools.lru_cache(maxsize=None)
def _build(n_tok, vocab, emb, n_pos, pos_w, n_bio, bio_w):
    out_w = emb + pos_w + bio_w
    pb_w = pos_w + bio_w
    chunk = 512
    assert n_tok % (NW * chunk * NBUF) == 0
    tok_per_w = n_tok // NW
    n_chunks = tok_per_w // chunk

    mesh = plsc.VectorSubcoreMesh(core_axis_name="c", subcore_axis_name="s")

    per_buf = [
        pltpu.VMEM((chunk,), jnp.int32),
        pltpu.VMEM((chunk,), jnp.int32),
        pltpu.VMEM((chunk,), jnp.int32),
        pltpu.VMEM((chunk, emb), jnp.float32),
        pltpu.VMEM((chunk, pb_w), jnp.float32),
        pltpu.SemaphoreType.DMA,
        pltpu.SemaphoreType.DMA,
        pltpu.SemaphoreType.DMA,
    ]

    @functools.partial(
        pl.kernel,
        mesh=mesh,
        out_type=jax.ShapeDtypeStruct((n_tok, out_w), jnp.float32),
        compiler_params=pltpu.CompilerParams(
            needs_layout_passes=False, use_tc_tiling_on_sc=False),
        scratch_types=[
            pltpu.VMEM((n_pos * pos_w,), jnp.float32),
            pltpu.VMEM((n_bio * bio_w,), jnp.float32),
        ] + per_buf * NBUF,
    )
    def sc_kernel(sent_hbm, f0_hbm, f1_hbm, pos_hbm, bio_hbm, table_hbm,
                  out_hbm, pos_v, bio_v, *bufs):
        wid = lax.axis_index("s") * NC + lax.axis_index("c")
        base = wid * tok_per_w
        pltpu.sync_copy(pos_hbm, pos_v)
        pltpu.sync_copy(bio_hbm, bio_v)
        iota = lax.iota(jnp.int32, LANES)
        nb = len(per_buf)

        def do_chunk(g, b, drain):
            idx_v, f0_v, f1_v, word_v, pb_v, gsem, wsem, psem = (
                bufs[b * nb:(b + 1) * nb])
            j = g * NBUF + b
            cb = base + j * chunk
            word_dst = out_hbm.at[pl.ds(cb, chunk), pl.ds(0, emb)]
            pb_dst = out_hbm.at[pl.ds(cb, chunk), pl.ds(emb, pb_w)]

            @pl.when(drain)
            def _():
                pltpu.make_async_copy(word_v, word_dst, wsem).wait()
                pltpu.make_async_copy(pb_v, pb_dst, psem).wait()

            pltpu.sync_copy(sent_hbm.at[pl.ds(cb, chunk)], idx_v)
            gather = pltpu.async_copy(table_hbm.at[idx_v], word_v, gsem)
            pltpu.sync_copy(f0_hbm.at[pl.ds(cb, chunk)], f0_v)
            pltpu.sync_copy(f1_hbm.at[pl.ds(cb, chunk)], f1_v)

            def fill_body(t, c):
                rows = t * LANES + iota
                f0s = f0_v[pl.ds(t * LANES, LANES)] * pos_w
                f1s = f1_v[pl.ds(t * LANES, LANES)] * bio_w
                for col in range(pos_w):
                    vals = plsc.load_gather(pos_v, [f0s + col])
                    plsc.store_scatter(
                        pb_v, [rows, jnp.full((LANES,), col, jnp.int32)],
                        vals)
                for col in range(bio_w):
                    vals = plsc.load_gather(bio_v, [f1s + col])
                    plsc.store_scatter(
                        pb_v,
                        [rows, jnp.full((LANES,), pos_w + col, jnp.int32)],
                        vals)
                return c

            lax.fori_loop(0, chunk // LANES, fill_body, 0)
            pltpu.async_copy(pb_v, pb_dst, psem)
            gather.wait()
            pltpu.async_copy(word_v, word_dst, wsem)

        def pair_body(g, carry):
            for b in range(NBUF):
                do_chunk(g, b, g > 0)
            return carry

        lax.fori_loop(0, n_chunks // NBUF, pair_body, 0)
        for b in range(NBUF):
            idx_v, f0_v, f1_v, word_v, pb_v, gsem, wsem, psem = (
                bufs[b * nb:(b + 1) * nb])
            j = n_chunks - NBUF + b
            cb = base + j * chunk
            pltpu.make_async_copy(
                word_v, out_hbm.at[pl.ds(cb, chunk), pl.ds(0, emb)],
                wsem).wait()
            pltpu.make_async_copy(
                pb_v, out_hbm.at[pl.ds(cb, chunk), pl.ds(emb, pb_w)],
                psem).wait()

    return sc_kernel


def kernel(sentence, features, embedding_matrix, pos_table, bio_table):
    b, l = sentence.shape
    vocab, emb = embedding_matrix.shape
    n_pos, pos_w = pos_table.shape
    n_bio, bio_w = bio_table.shape
    n_tok = b * l
    sent = sentence.reshape(n_tok)
    f0 = features[..., 0].reshape(n_tok)
    f1 = features[..., 1].reshape(n_tok)
    fn = _build(n_tok, vocab, emb, n_pos, pos_w, n_bio, bio_w)
    out = fn(sent, f0, f1, pos_table.reshape(n_pos * pos_w),
             bio_table.reshape(n_bio * bio_w), embedding_matrix)
    return out.reshape(b, l, emb + pos_w + bio_w)
